# baseline (speedup 1.0000x reference)
import jax
import jax.numpy as jnp
from jax.experimental import pallas as pl
from jax.experimental.pallas import tpu as pltpu

_K = 32


def _enc_topk_kernel(x_ref, w_ref, z_ref, zs_ref, cnt_ref):
    z = jax.lax.dot_general(
        x_ref[...], w_ref[...], (((1,), (1,)), ((), ())),
        preferred_element_type=jnp.float32)
    z = jnp.maximum(z, 0.0)
    z_ref[...] = z
    del z

    zfirst = z_ref[...]
    kf = float(_K)
    pos = jnp.sum((zfirst > 0.0).astype(jnp.float32), axis=1, keepdims=True)
    hi0 = jnp.max(zfirst, axis=1, keepdims=True) + 1.0
    del zfirst
    lo0 = jnp.zeros_like(hi0)
    found0 = jnp.where(pos <= kf, 1.0, 0.0)
    t0 = jnp.zeros_like(hi0)

    clo0 = pos

    def cond(state):
        i, lo, hi, clo, t, found = state
        return jnp.logical_and(i < 26, jnp.min(found) < 0.5)

    def body(state):
        i, lo, hi, clo, t, found = state
        third = (hi - lo) * (1.0 / 3.0)
        m1 = lo + third
        m2 = hi - third
        zc = z_ref[...]
        b1 = (zc >= m1).astype(jnp.float32)
        b2 = (zc >= m2).astype(jnp.float32)
        c1 = jnp.sum(b1, axis=1, keepdims=True)
        c2 = jnp.sum(b2, axis=1, keepdims=True)
        hit1 = c1 == kf
        hit2 = c2 == kf
        hit = jnp.where(jnp.logical_or(hit1, hit2), 1.0, 0.0) * (1.0 - found)
        t = jnp.where(hit > 0.5, jnp.where(hit1, m1, m2), t)
        go = (1.0 - found) * (1.0 - hit) > 0.5
        lo_n = jnp.where(c2 > kf, m2, jnp.where(c1 > kf, m1, lo))
        clo_n = jnp.where(c2 > kf, c2, jnp.where(c1 > kf, c1, clo))
        hi_n = jnp.where(c1 < kf, m1, jnp.where(c2 < kf, m2, hi))
        lo = jnp.where(go, lo_n, lo)
        clo = jnp.where(go, clo_n, clo)
        hi = jnp.where(go, hi_n, hi)
        found = jnp.maximum(found, hit)
        return (i + 1, lo, hi, clo, t, found)

    _, lo, hi, clo, t, found = jax.lax.while_loop(
        cond, body, (jnp.int32(0), lo0, hi0, clo0, t0, found0))
    thresh = jnp.where(found > 0.5, t, lo)
    cnt = jnp.where(pos <= kf, pos, jnp.where(found > 0.5, kf, clo))

    zfull = z_ref[...]
    zs_ref[...] = jnp.where(zfull >= thresh, zfull, 0.0)
    cnt_ref[...] = cnt


def _dec_kernel(zs_ref, w_ref, cnt_ref, xhat_ref, act_ref):
    xhat_ref[...] = jax.lax.dot_general(
        zs_ref[...], w_ref[...], (((1,), (1,)), ((), ())),
        preferred_element_type=jnp.float32)
    t = pl.program_id(0)

    @pl.when(t == 0)
    def _():
        act_ref[...] = jnp.zeros_like(act_ref)

    act_ref[...] = act_ref[...] + jnp.sum(cnt_ref[...]).reshape(1, 1)


@jax.jit
def kernel(x, W_enc, W_dec):
    n_tokens, input_dim = x.shape
    hidden_dim = W_enc.shape[0]
    bt1 = 64
    bt = 128
    n_blocks = n_tokens // bt

    z, z_sparse, counts = pl.pallas_call(
        _enc_topk_kernel,
        grid=(n_tokens // bt1,),
        in_specs=[
            pl.BlockSpec((bt1, input_dim), lambda t: (t, 0)),
            pl.BlockSpec((hidden_dim, input_dim), lambda t: (0, 0)),
        ],
        out_specs=[
            pl.BlockSpec((bt1, hidden_dim), lambda t: (t, 0)),
            pl.BlockSpec((bt1, hidden_dim), lambda t: (t, 0)),
            pl.BlockSpec((bt1, 1), lambda t: (t, 0)),
        ],
        out_shape=[
            jax.ShapeDtypeStruct((n_tokens, hidden_dim), jnp.float32),
            jax.ShapeDtypeStruct((n_tokens, hidden_dim), jnp.float32),
            jax.ShapeDtypeStruct((n_tokens, 1), jnp.float32),
        ],
        compiler_params=pltpu.CompilerParams(
            dimension_semantics=("arbitrary",)),
    )(x, W_enc)

    x_hat, act_sum = pl.pallas_call(
        _dec_kernel,
        grid=(n_blocks,),
        in_specs=[
            pl.BlockSpec((bt, hidden_dim), lambda t: (t, 0)),
            pl.BlockSpec((input_dim, hidden_dim), lambda t: (0, 0)),
            pl.BlockSpec((bt, 1), lambda t: (t, 0)),
        ],
        out_specs=[
            pl.BlockSpec((bt, input_dim), lambda t: (t, 0)),
            pl.BlockSpec((1, 1), lambda t: (0, 0)),
        ],
        out_shape=[
            jax.ShapeDtypeStruct((n_tokens, input_dim), jnp.float32),
            jax.ShapeDtypeStruct((1, 1), jnp.float32),
        ],
        compiler_params=pltpu.CompilerParams(
            dimension_semantics=("arbitrary",)),
    )(z_sparse, W_dec, counts)

    active_features = act_sum[0, 0] / n_tokens
    return (x_hat, z_sparse, z, active_features)
